# baseline (speedup 1.0000x reference)
import jax
import jax.numpy as jnp
from jax import lax
from jax.experimental import pallas as pl
from jax.experimental.pallas import tpu as pltpu
from jax.experimental.pallas import tpu_sc as plsc

_BLOCK_ROWS = 256
_LANES = 16
_NUM_CORES = 2
_NUM_SUBCORES = 16
_NW = _NUM_CORES * _NUM_SUBCORES


def _copy_block(x_ref, o_ref):
    o_ref[...] = x_ref[...]


def _tc_copy(x):
    n_rows, n_cols = x.shape
    return pl.pallas_call(
        _copy_block,
        grid=(n_rows // _BLOCK_ROWS,),
        in_specs=[pl.BlockSpec((_BLOCK_ROWS, n_cols), lambda i: (i, 0))],
        out_specs=pl.BlockSpec((_BLOCK_ROWS, n_cols), lambda i: (i, 0)),
        out_shape=jax.ShapeDtypeStruct(x.shape, x.dtype),
        compiler_params=pltpu.CompilerParams(
            dimension_semantics=("parallel",),
        ),
    )(x)


_ROW = 128


def _make_sc_diag(n, dtype):
    per_w = n // _NW
    mesh = plsc.VectorSubcoreMesh(
        core_axis_name="c", subcore_axis_name="s"
    )

    def body(fill_hbm, idx_hbm, y_ref, fill_v, idx_v, rows_v, sem):
        wid = lax.axis_index("s") * _NUM_CORES + lax.axis_index("c")
        base = wid * per_w
        pltpu.sync_copy(fill_hbm, fill_v)
        pltpu.sync_copy(idx_hbm.at[pl.ds(base, per_w)], idx_v)
        pltpu.async_copy(y_ref.at[idx_v], rows_v, sem).wait()
        fv = fill_v[...]
        lane = lax.iota(jnp.int32, _LANES)

        def fix_row(j, carry):
            s = ((j % _ROW) // _LANES) * _LANES
            sl = pl.ds(s, _LANES)
            rows_v[j, sl] = jnp.where(lane == j % _LANES, fv, rows_v[j, sl])
            return carry

        lax.fori_loop(0, per_w, fix_row, 0)
        pltpu.async_copy(rows_v, y_ref.at[idx_v], sem).wait()

    return pl.kernel(
        body,
        out_type=(),
        mesh=mesh,
        scratch_types=[
            pltpu.VMEM((_LANES,), dtype),
            pltpu.VMEM((per_w,), jnp.int32),
            pltpu.VMEM((per_w, _ROW), dtype),
            pltpu.SemaphoreType.DMA,
        ],
    )


def kernel(x, fill_value):
    n = min(x.shape)
    y = _tc_copy(x)
    fill_arr = jnp.full((_LANES,), fill_value, x.dtype)
    d = jnp.arange(n, dtype=jnp.int32)
    row_idx = d * (n // _ROW) + d // _ROW
    y_ref = jax.new_ref(y.reshape(n * n // _ROW, _ROW))
    _make_sc_diag(n, x.dtype)(fill_arr, row_idx, y_ref)
    return y_ref[...].reshape(n, n)


# speedup vs baseline: 4.0466x; 4.0466x over previous
import jax
import jax.numpy as jnp
from jax import lax
from jax.experimental import pallas as pl
from jax.experimental.pallas import tpu as pltpu
from jax.experimental.pallas import tpu_sc as plsc

_BLOCK_ROWS = 256
_LANES = 16
_NUM_CORES = 2
_NUM_SUBCORES = 16
_NW = _NUM_CORES * _NUM_SUBCORES
_DBLK = 128


def _copy_block(x_ref, o_ref):
    o_ref[...] = x_ref[...]


def _tc_copy(x):
    n_rows, n_cols = x.shape
    return pl.pallas_call(
        _copy_block,
        grid=(n_rows // _BLOCK_ROWS,),
        in_specs=[pl.BlockSpec((_BLOCK_ROWS, n_cols), lambda i: (i, 0))],
        out_specs=pl.BlockSpec((_BLOCK_ROWS, n_cols), lambda i: (i, 0)),
        out_shape=jax.ShapeDtypeStruct(x.shape, x.dtype),
        compiler_params=pltpu.CompilerParams(
            dimension_semantics=("parallel",),
        ),
    )(x)


def _make_sc_diag(n, dtype):
    blocks_per_w = n // _DBLK // _NW
    mesh = plsc.VectorSubcoreMesh(
        core_axis_name="c", subcore_axis_name="s"
    )

    def body(fill_hbm, y_ref, fill_v, blk_v, sem):
        wid = lax.axis_index("s") * _NUM_CORES + lax.axis_index("c")
        base = wid * (blocks_per_w * _DBLK)
        pltpu.sync_copy(fill_hbm, fill_v)
        for t in range(blocks_per_w):
            b = base + t * _DBLK
            pltpu.make_async_copy(
                y_ref.at[pl.ds(b, _DBLK), pl.ds(b, _DBLK)],
                blk_v.at[t],
                sem,
            ).start()
        for t in range(blocks_per_w):
            b = base + t * _DBLK
            pltpu.make_async_copy(
                y_ref.at[pl.ds(b, _DBLK), pl.ds(b, _DBLK)],
                blk_v.at[t],
                sem,
            ).wait()
        fv = fill_v[...]
        lane = lax.iota(jnp.int32, _LANES)
        for t in range(blocks_per_w):
            def fix_row(k, carry, t=t):
                sl = pl.ds((k // _LANES) * _LANES, _LANES)
                blk_v[t, k, sl] = jnp.where(
                    lane == k % _LANES, fv, blk_v[t, k, sl]
                )
                return carry

            lax.fori_loop(0, _DBLK, fix_row, 0)
        for t in range(blocks_per_w):
            b = base + t * _DBLK
            pltpu.make_async_copy(
                blk_v.at[t],
                y_ref.at[pl.ds(b, _DBLK), pl.ds(b, _DBLK)],
                sem,
            ).start()
        for t in range(blocks_per_w):
            b = base + t * _DBLK
            pltpu.make_async_copy(
                blk_v.at[t],
                y_ref.at[pl.ds(b, _DBLK), pl.ds(b, _DBLK)],
                sem,
            ).wait()

    return pl.kernel(
        body,
        out_type=(),
        mesh=mesh,
        scratch_types=[
            pltpu.VMEM((_LANES,), dtype),
            pltpu.VMEM((blocks_per_w, _DBLK, _DBLK), dtype),
            pltpu.SemaphoreType.DMA,
        ],
    )


def kernel(x, fill_value):
    n = min(x.shape)
    y = _tc_copy(x)
    fill_arr = jnp.full((_LANES,), fill_value, x.dtype)
    y_ref = jax.new_ref(y)
    _make_sc_diag(n, x.dtype)(fill_arr, y_ref)
    return y_ref[...]


# speedup vs baseline: 4.0689x; 1.0055x over previous
import jax
import jax.numpy as jnp
from jax import lax
from jax.experimental import pallas as pl
from jax.experimental.pallas import tpu as pltpu
from jax.experimental.pallas import tpu_sc as plsc

_BLOCK_ROWS = 256
_LANES = 16
_NUM_CORES = 2
_NUM_SUBCORES = 16
_NW = _NUM_CORES * _NUM_SUBCORES
_DBLK = 128


def _copy_block(x_ref, o_ref):
    o_ref[...] = x_ref[...]


def _tc_copy(x):
    n_rows, n_cols = x.shape
    return pl.pallas_call(
        _copy_block,
        grid=(n_rows // _BLOCK_ROWS,),
        in_specs=[pl.BlockSpec((_BLOCK_ROWS, n_cols), lambda i: (i, 0))],
        out_specs=pl.BlockSpec((_BLOCK_ROWS, n_cols), lambda i: (i, 0)),
        out_shape=jax.ShapeDtypeStruct(x.shape, x.dtype),
        compiler_params=pltpu.CompilerParams(
            dimension_semantics=("parallel",),
        ),
    )(x)


def _make_sc_diag(n, dtype):
    blocks_per_w = n // _DBLK // _NW
    mesh = plsc.VectorSubcoreMesh(
        core_axis_name="c", subcore_axis_name="s"
    )

    def body(fill_hbm, y_ref, fill_v, blk_v, sem):
        wid = lax.axis_index("s") * _NUM_CORES + lax.axis_index("c")
        base = wid * (blocks_per_w * _DBLK)
        pltpu.sync_copy(fill_hbm, fill_v)
        for t in range(blocks_per_w):
            b = base + t * _DBLK
            pltpu.make_async_copy(
                y_ref.at[pl.ds(b, _DBLK), pl.ds(b, _DBLK)],
                blk_v.at[t],
                sem,
            ).start()
        for t in range(blocks_per_w):
            b = base + t * _DBLK
            pltpu.make_async_copy(
                y_ref.at[pl.ds(b, _DBLK), pl.ds(b, _DBLK)],
                blk_v.at[t],
                sem,
            ).wait()
        fv = fill_v[...]
        lane = lax.iota(jnp.int32, _LANES)
        for t in range(blocks_per_w):
            for g in range(_DBLK // _LANES):
                idxv = lane + g * _LANES
                plsc.store_scatter(blk_v.at[t], (idxv, idxv), fv)
        for t in range(blocks_per_w):
            b = base + t * _DBLK
            pltpu.make_async_copy(
                blk_v.at[t],
                y_ref.at[pl.ds(b, _DBLK), pl.ds(b, _DBLK)],
                sem,
            ).start()
        for t in range(blocks_per_w):
            b = base + t * _DBLK
            pltpu.make_async_copy(
                blk_v.at[t],
                y_ref.at[pl.ds(b, _DBLK), pl.ds(b, _DBLK)],
                sem,
            ).wait()

    return pl.kernel(
        body,
        out_type=(),
        mesh=mesh,
        compiler_params=pltpu.CompilerParams(needs_layout_passes=False),
        scratch_types=[
            pltpu.VMEM((_LANES,), dtype),
            pltpu.VMEM((blocks_per_w, _DBLK, _DBLK), dtype),
            pltpu.SemaphoreType.DMA,
        ],
    )


def kernel(x, fill_value):
    n = min(x.shape)
    y = _tc_copy(x)
    fill_arr = jnp.full((_LANES,), fill_value, x.dtype)
    y_ref = jax.new_ref(y)
    _make_sc_diag(n, x.dtype)(fill_arr, y_ref)
    return y_ref[...]
